# baseline (speedup 1.0000x reference)
import jax
import jax.numpy as jnp
from jax.experimental import pallas as pl
from jax.experimental.pallas import tpu as pltpu

N = 4096
D = 64
BM = 128
NB = N // BM
QC_RES = 2944
CQ_RES = 2944
QC_NBR = QC_RES // BM
CQ_NBR = CQ_RES // BM


def _phase_kernel(x0_ref, qc_ref, cq_ref, out_ref,
                  qc16_ref, cq16_ref, msg_ref, xcur_ref):
    p = pl.program_id(0)
    j = pl.program_id(1)
    rows = pl.ds(j * BM, BM)

    @pl.when(p == 0)
    def _():
        qc16 = qc_ref[...].astype(jnp.bfloat16)

        @pl.when(j < QC_NBR)
        def _():
            qc16_ref[rows, :] = qc16

        x0b = x0_ref[...].astype(jnp.bfloat16)
        msg_ref[rows, :] = jnp.dot(
            qc16, x0b, preferred_element_type=jnp.float32).astype(jnp.bfloat16)
        out_ref[rows, :] = x0_ref[rows, :]

    @pl.when(p == 1)
    def _():
        cq16 = cq_ref[...].astype(jnp.bfloat16)

        @pl.when(j < CQ_NBR)
        def _():
            cq16_ref[rows, :] = cq16

        t = jnp.dot(cq16, msg_ref[...],
                    preferred_element_type=jnp.float32) + x0_ref[rows, :]
        xcur_ref[rows, :] = t
        out_ref[rows, :] += t

    @pl.when((p == 2) | (p == 4))
    def _():
        x16 = xcur_ref[...].astype(jnp.bfloat16)
        res = jax.lax.cond(
            j < QC_NBR,
            lambda: jnp.dot(qc16_ref[rows, :], x16,
                            preferred_element_type=jnp.float32),
            lambda: jnp.dot(qc_ref[...].astype(jnp.bfloat16), x16,
                            preferred_element_type=jnp.float32),
        )
        msg_ref[rows, :] = res.astype(jnp.bfloat16)

    @pl.when((p == 3) | (p == 5))
    def _():
        res = jax.lax.cond(
            j < CQ_NBR,
            lambda: jnp.dot(cq16_ref[rows, :], msg_ref[...],
                            preferred_element_type=jnp.float32),
            lambda: jnp.dot(cq_ref[...].astype(jnp.bfloat16), msg_ref[...],
                            preferred_element_type=jnp.float32),
        )
        t = res + xcur_ref[rows, :]

        @pl.when(p == 3)
        def _():
            xcur_ref[rows, :] = t
            out_ref[rows, :] += t

        @pl.when(p == 5)
        def _():
            out_ref[rows, :] = (out_ref[rows, :] + t) * 0.25


def _qc_index(p, j):
    return (jnp.where(p == 0, j,
                      jnp.where((p == 2) | (p == 4),
                                jnp.maximum(j, QC_NBR), NB - 1)), 0)


def _cq_index(p, j):
    return (jnp.where(p == 1, j,
                      jnp.where((p == 3) | (p == 5),
                                jnp.maximum(j, CQ_NBR),
                                jnp.where(p == 0, 0, NB - 1))), 0)


def kernel(skill_embs, HG_qc, HG_cq):
    return pl.pallas_call(
        _phase_kernel,
        grid=(6, NB),
        in_specs=[
            pl.BlockSpec((N, D), lambda p, j: (0, 0)),
            pl.BlockSpec((BM, N), _qc_index),
            pl.BlockSpec((BM, N), _cq_index),
        ],
        out_specs=pl.BlockSpec((N, D), lambda p, j: (0, 0)),
        out_shape=jax.ShapeDtypeStruct((N, D), jnp.float32),
        compiler_params=pltpu.CompilerParams(vmem_limit_bytes=66584576),
        scratch_shapes=[
            pltpu.VMEM((QC_RES, N), jnp.bfloat16),
            pltpu.VMEM((CQ_RES, N), jnp.bfloat16),
            pltpu.VMEM((N, D), jnp.bfloat16),
            pltpu.VMEM((N, D), jnp.float32),
        ],
    )(skill_embs, HG_qc, HG_cq)


# speedup vs baseline: 1.2964x; 1.2964x over previous
import jax
import jax.numpy as jnp
from jax.experimental import pallas as pl
from jax.experimental.pallas import tpu as pltpu

N = 4096
D = 64
BM = 256
NB = N // BM
QC_RES = 2560
CQ_RES = 2048
QC_NBR = QC_RES // BM
CQ_NBR = CQ_RES // BM


def _phase_kernel(x0_ref, qc_ref, cq_ref, out_ref,
                  qc16_ref, cq16_ref, msg_ref, xcur_ref):
    p = pl.program_id(0)
    j = pl.program_id(1)
    rows = pl.ds(j * BM, BM)

    @pl.when(p == 0)
    def _():
        qc16 = qc_ref[...].astype(jnp.bfloat16)

        @pl.when(j < QC_NBR)
        def _():
            qc16_ref[rows, :] = qc16

        x0b = x0_ref[...].astype(jnp.bfloat16)
        msg_ref[rows, :] = jnp.dot(
            qc16, x0b, preferred_element_type=jnp.float32).astype(jnp.bfloat16)
        out_ref[rows, :] = x0_ref[rows, :]

    @pl.when(p == 1)
    def _():
        cq16 = cq_ref[...].astype(jnp.bfloat16)

        @pl.when(j < CQ_NBR)
        def _():
            cq16_ref[rows, :] = cq16

        t = jnp.dot(cq16, msg_ref[...],
                    preferred_element_type=jnp.float32) + x0_ref[rows, :]
        xcur_ref[rows, :] = t
        out_ref[rows, :] += t

    is_qc_phase = (p == 2) | (p == 4)

    @pl.when(is_qc_phase & (j < QC_NBR))
    def _():
        x16 = xcur_ref[...].astype(jnp.bfloat16)
        msg_ref[rows, :] = jnp.dot(
            qc16_ref[rows, :], x16,
            preferred_element_type=jnp.float32).astype(jnp.bfloat16)

    @pl.when(is_qc_phase & (j >= QC_NBR))
    def _():
        x16 = xcur_ref[...].astype(jnp.bfloat16)
        msg_ref[rows, :] = jnp.dot(
            qc_ref[...].astype(jnp.bfloat16), x16,
            preferred_element_type=jnp.float32).astype(jnp.bfloat16)

    def _cq_phase_epilogue(t):
        @pl.when(p == 3)
        def _():
            xcur_ref[rows, :] = t
            out_ref[rows, :] += t

        @pl.when(p == 5)
        def _():
            out_ref[rows, :] = (out_ref[rows, :] + t) * 0.25

    is_cq_phase = (p == 3) | (p == 5)

    @pl.when(is_cq_phase & (j < CQ_NBR))
    def _():
        t = jnp.dot(cq16_ref[rows, :], msg_ref[...],
                    preferred_element_type=jnp.float32) + xcur_ref[rows, :]
        _cq_phase_epilogue(t)

    @pl.when(is_cq_phase & (j >= CQ_NBR))
    def _():
        t = jnp.dot(cq_ref[...].astype(jnp.bfloat16), msg_ref[...],
                    preferred_element_type=jnp.float32) + xcur_ref[rows, :]
        _cq_phase_epilogue(t)


def _qc_index(p, j):
    return (jnp.where(p == 0, j,
                      jnp.where((p == 2) | (p == 4),
                                jnp.maximum(j, QC_NBR), NB - 1)), 0)


def _cq_index(p, j):
    return (jnp.where(p == 1, j,
                      jnp.where((p == 3) | (p == 5),
                                jnp.maximum(j, CQ_NBR),
                                jnp.where(p == 0, 0, NB - 1))), 0)


def kernel(skill_embs, HG_qc, HG_cq):
    return pl.pallas_call(
        _phase_kernel,
        grid=(6, NB),
        in_specs=[
            pl.BlockSpec((N, D), lambda p, j: (0, 0)),
            pl.BlockSpec((BM, N), _qc_index),
            pl.BlockSpec((BM, N), _cq_index),
        ],
        out_specs=pl.BlockSpec((N, D), lambda p, j: (0, 0)),
        out_shape=jax.ShapeDtypeStruct((N, D), jnp.float32),
        compiler_params=pltpu.CompilerParams(vmem_limit_bytes=66584576),
        scratch_shapes=[
            pltpu.VMEM((QC_RES, N), jnp.bfloat16),
            pltpu.VMEM((CQ_RES, N), jnp.bfloat16),
            pltpu.VMEM((N, D), jnp.bfloat16),
            pltpu.VMEM((N, D), jnp.float32),
        ],
    )(skill_embs, HG_qc, HG_cq)


# speedup vs baseline: 1.7627x; 1.3597x over previous
import jax
import jax.numpy as jnp
from jax import lax
from jax.experimental import pallas as pl
from jax.experimental.pallas import tpu as pltpu

N = 4096
D = 64
BM = 256
NB = N // BM
QC_RES = 2560
CQ_RES = 2304
QNB = QC_RES // BM
CNB = CQ_RES // BM
QTL = NB - QNB
CTL = NB - CNB
NR = 3

S1 = NB
S2 = S1 + NB
S3 = S2 + QTL
S4 = S3 + CTL
S5 = S4 + QTL
TOT = S5 + CTL


def _kernel(x0_ref, qc_ref, cq_ref, out_ref,
            qc16_ref, cq16_ref, ring_ref, msg_ref, xcur_ref, x16_ref,
            sem_ref):

    def issue(i):
        @pl.when(i < TOT)
        def _():
            is_qc = (i < S1) | ((i >= S2) & (i < S3)) | ((i >= S4) & (i < S5))
            blk = jnp.where(i < S1, i,
                  jnp.where(i < S2, i - S1,
                  jnp.where(i < S3, i - S2 + QNB,
                  jnp.where(i < S4, i - S3 + CNB,
                  jnp.where(i < S5, i - S4 + QNB, i - S5 + CNB)))))
            slot = lax.rem(i, NR)

            @pl.when(is_qc)
            def _():
                pltpu.make_async_copy(qc_ref.at[pl.ds(blk * BM, BM), :],
                                      ring_ref.at[slot],
                                      sem_ref.at[slot]).start()

            @pl.when(jnp.logical_not(is_qc))
            def _():
                pltpu.make_async_copy(cq_ref.at[pl.ds(blk * BM, BM), :],
                                      ring_ref.at[slot],
                                      sem_ref.at[slot]).start()

    def wait(slot):
        pltpu.make_async_copy(qc_ref.at[pl.ds(0, BM), :],
                              ring_ref.at[slot], sem_ref.at[slot]).wait()

    x16_ref[...] = x0_ref[...].astype(jnp.bfloat16)
    for i in range(NR):
        issue(jnp.int32(i))

    def p0(j, _):
        slot = lax.rem(j, NR)
        wait(slot)
        b16 = ring_ref[slot].astype(jnp.bfloat16)
        rows = pl.ds(j * BM, BM)

        @pl.when(j < QNB)
        def _():
            qc16_ref[rows, :] = b16

        msg_ref[rows, :] = jnp.dot(
            b16, x16_ref[...],
            preferred_element_type=jnp.float32).astype(jnp.bfloat16)
        out_ref[rows, :] = x0_ref[rows, :]
        issue(j + NR)
        return 0

    lax.fori_loop(0, NB, p0, 0)

    def p1(j, _):
        slot = lax.rem(S1 + j, NR)
        wait(slot)
        b16 = ring_ref[slot].astype(jnp.bfloat16)
        rows = pl.ds(j * BM, BM)

        @pl.when(j < CNB)
        def _():
            cq16_ref[rows, :] = b16

        t = jnp.dot(b16, msg_ref[...],
                    preferred_element_type=jnp.float32) + x0_ref[rows, :]
        xcur_ref[rows, :] = t
        out_ref[rows, :] += t
        issue(S1 + j + NR)
        return 0

    lax.fori_loop(0, NB, p1, 0)

    def qc_phase(base):
        x16_ref[...] = xcur_ref[...].astype(jnp.bfloat16)

        def res(j, _):
            rows = pl.ds(j * BM, BM)
            msg_ref[rows, :] = jnp.dot(
                qc16_ref[rows, :], x16_ref[...],
                preferred_element_type=jnp.float32).astype(jnp.bfloat16)
            return 0

        lax.fori_loop(0, QNB, res, 0)

        def tail(k, _):
            i = base + k
            slot = lax.rem(i, NR)
            wait(slot)
            b16 = ring_ref[slot].astype(jnp.bfloat16)
            rows = pl.ds((QNB + k) * BM, BM)
            msg_ref[rows, :] = jnp.dot(
                b16, x16_ref[...],
                preferred_element_type=jnp.float32).astype(jnp.bfloat16)
            issue(i + NR)
            return 0

        lax.fori_loop(0, QTL, tail, 0)

    def cq_phase(base, last):
        def epi(rows, t):
            if last:
                out_ref[rows, :] = (out_ref[rows, :] + t) * 0.25
            else:
                xcur_ref[rows, :] = t
                out_ref[rows, :] += t

        def res(j, _):
            rows = pl.ds(j * BM, BM)
            t = jnp.dot(cq16_ref[rows, :], msg_ref[...],
                        preferred_element_type=jnp.float32) + xcur_ref[rows, :]
            epi(rows, t)
            return 0

        lax.fori_loop(0, CNB, res, 0)

        def tail(k, _):
            i = base + k
            slot = lax.rem(i, NR)
            wait(slot)
            b16 = ring_ref[slot].astype(jnp.bfloat16)
            rows = pl.ds((CNB + k) * BM, BM)
            t = jnp.dot(b16, msg_ref[...],
                        preferred_element_type=jnp.float32) + xcur_ref[rows, :]
            epi(rows, t)
            issue(i + NR)
            return 0

        lax.fori_loop(0, CTL, tail, 0)

    qc_phase(S2)
    cq_phase(S3, False)
    qc_phase(S4)
    cq_phase(S5, True)


def kernel(skill_embs, HG_qc, HG_cq):
    return pl.pallas_call(
        _kernel,
        in_specs=[
            pl.BlockSpec(memory_space=pltpu.MemorySpace.VMEM),
            pl.BlockSpec(memory_space=pltpu.MemorySpace.HBM),
            pl.BlockSpec(memory_space=pltpu.MemorySpace.HBM),
        ],
        out_specs=pl.BlockSpec(memory_space=pltpu.MemorySpace.VMEM),
        out_shape=jax.ShapeDtypeStruct((N, D), jnp.float32),
        compiler_params=pltpu.CompilerParams(vmem_limit_bytes=66584576),
        scratch_shapes=[
            pltpu.VMEM((QC_RES, N), jnp.bfloat16),
            pltpu.VMEM((CQ_RES, N), jnp.bfloat16),
            pltpu.VMEM((NR, BM, N), jnp.float32),
            pltpu.VMEM((N, D), jnp.bfloat16),
            pltpu.VMEM((N, D), jnp.float32),
            pltpu.VMEM((N, D), jnp.bfloat16),
            pltpu.SemaphoreType.DMA((NR,)),
        ],
    )(skill_embs, HG_qc, HG_cq)
